# baseline (speedup 1.0000x reference)
import functools

import jax
import jax.numpy as jnp
from jax import lax
from jax.experimental import pallas as pl
from jax.experimental.pallas import tpu as pltpu
from jax.experimental.pallas import tpu_sc as plsc

B, C, H, W = 16, 3, 512, 512
N = B * H * W
NSUB = 32
PER_SUB = N // NSUB
CHUNK = 8192
NCHUNK = PER_SUB // CHUNK
L = 16
UNROLL = 8
NHIST = 4
NB = 1024
SHIFT1 = 20
SHIFT2 = 10
H1_STRIDE = NB * L
H2_BINS = 1032
H2_STRIDE = H2_BINS * L
R_HIGH = 10486
R_LOW = 10487

_mesh = plsc.VectorSubcoreMesh(
    core_axis_name="c", subcore_axis_name="s", num_cores=2, num_subcores=16
)
_sc_params = pltpu.CompilerParams(needs_layout_passes=False)


def _gray_body(img_ref, out_ref):
  r = img_ref[0, 0]
  g = img_ref[0, 1]
  b = img_ref[0, 2]
  out_ref[...] = (0.2989 * r + 0.587 * g + 0.114 * b).reshape(H * W)


def _grayscale(img):
  return pl.pallas_call(
      _gray_body,
      out_shape=jax.ShapeDtypeStruct((N,), jnp.float32),
      grid=(B,),
      in_specs=[pl.BlockSpec((1, C, H, W), lambda i: (i, 0, 0, 0))],
      out_specs=pl.BlockSpec((H * W,), lambda i: (i,)),
  )(img)


def _zero_hist(hist, nwords):
  zeros = jnp.zeros((L,), jnp.int32)

  @plsc.parallel_loop(0, nwords, step=L * UNROLL, unroll=4)
  def _(i):
    for u in range(UNROLL):
      hist[pl.ds(i + u * L, L)] = zeros


def _reduce_hists(hist, out, stride, nwords):
  @plsc.parallel_loop(0, nwords, step=L, unroll=4)
  def _(i):
    acc = hist[pl.ds(i, L)]
    for h in range(1, NHIST):
      acc = acc + hist[pl.ds(h * stride + i, L)]
    out[pl.ds(i, L)] = acc


def _rank_scan(h, target):
  def cond(c):
    return c[1] < target

  def body(c):
    blk = c[0] - 1
    return blk, c[1] + jnp.sum(h[pl.ds(blk * L, L)])

  blk, acc = lax.while_loop(cond, body, (jnp.int32(NB // L), jnp.int32(0)))
  v = h[pl.ds(blk * L, L)]
  s_blk = jnp.sum(v)
  acc0 = acc - s_blk
  suf = (s_blk - jnp.cumsum(v)) + v
  mask = (acc0 + suf) >= target
  nsel = plsc.all_reduce_population_count(mask)
  b_vec = blk * L + nsel - 1
  above = acc0 + jnp.sum(jnp.where(mask, 0, v))
  return b_vec, above


def _stream_chunks(gray_hbm, base, buf0, buf1, sem0, sem1, process, carry):
  pltpu.async_copy(gray_hbm.at[pl.ds(base, CHUNK)], buf0, sem0)
  pltpu.async_copy(gray_hbm.at[pl.ds(base + CHUNK, CHUNK)], buf1, sem1)

  def wait(buf, sem):
    pltpu.make_async_copy(gray_hbm.at[pl.ds(0, CHUNK)], buf, sem).wait()

  def body(i, c2):
    c = 2 * i
    wait(buf0, sem0)
    c2 = process(buf0, c2)
    pltpu.async_copy(
        gray_hbm.at[pl.ds(base + (c + 2) * CHUNK, CHUNK)], buf0, sem0)
    wait(buf1, sem1)
    c2 = process(buf1, c2)
    pltpu.async_copy(
        gray_hbm.at[pl.ds(base + (c + 3) * CHUNK, CHUNK)], buf1, sem1)
    return c2

  carry = lax.fori_loop(0, NCHUNK // 2 - 1, body, carry)
  wait(buf0, sem0)
  carry = process(buf0, carry)
  wait(buf1, sem1)
  carry = process(buf1, carry)
  return carry


@functools.partial(
    pl.kernel,
    mesh=_mesh,
    out_type=jax.ShapeDtypeStruct((NSUB, H1_STRIDE), jnp.int32),
    scratch_types=[
        pltpu.VMEM((CHUNK,), jnp.float32),
        pltpu.VMEM((CHUNK,), jnp.float32),
        pltpu.VMEM((NHIST * H1_STRIDE,), jnp.int32),
        pltpu.VMEM((H1_STRIDE,), jnp.int32),
        pltpu.SemaphoreType.DMA,
        pltpu.SemaphoreType.DMA,
    ],
    compiler_params=_sc_params,
)
def _sc_round1(gray_hbm, out_hbm, buf0, buf1, hist, red, sem0, sem1):
  wid = lax.axis_index("s") * 2 + lax.axis_index("c")
  _zero_hist(hist, NHIST * H1_STRIDE)
  lane = lax.iota(jnp.int32, L)
  lanes = [lane + h * H1_STRIDE for h in range(NHIST)]
  ones = jnp.ones((L,), jnp.int32)

  def process(buf, carry):
    @plsc.parallel_loop(0, CHUNK, step=L * NHIST, unroll=2)
    def _(i):
      for h in range(NHIST):
        x = buf[pl.ds(i + h * L, L)]
        bits = plsc.bitcast(x, jnp.int32)
        idx = lax.shift_left(
            lax.shift_right_logical(bits, SHIFT1), 4) + lanes[h]
        plsc.addupdate_scatter(hist, [idx], ones)

    return carry

  _stream_chunks(gray_hbm, wid * PER_SUB, buf0, buf1, sem0, sem1, process, 0)
  _reduce_hists(hist, red, H1_STRIDE, H1_STRIDE)
  pltpu.sync_copy(red, out_hbm.at[wid])


@functools.partial(
    pl.kernel,
    mesh=_mesh,
    out_type=(
        jax.ShapeDtypeStruct((NSUB, H2_STRIDE), jnp.int32),
        jax.ShapeDtypeStruct((NSUB, 2 * L), jnp.int32),
    ),
    scratch_types=[
        pltpu.VMEM((CHUNK,), jnp.float32),
        pltpu.VMEM((CHUNK,), jnp.float32),
        pltpu.VMEM((NB,), jnp.int32),
        pltpu.VMEM((NHIST * H2_STRIDE,), jnp.int32),
        pltpu.VMEM((H2_STRIDE,), jnp.int32),
        pltpu.VMEM((2 * L,), jnp.int32),
        pltpu.SemaphoreType.DMA,
        pltpu.SemaphoreType.DMA,
    ],
    compiler_params=_sc_params,
)
def _sc_round2(gray_hbm, h1_hbm, out_hbm, meta_hbm,
               buf0, buf1, h1v, hist, red, metav, sem0, sem1):
  wid = lax.axis_index("s") * 2 + lax.axis_index("c")
  pltpu.sync_copy(h1_hbm, h1v)
  _zero_hist(hist, NHIST * H2_STRIDE)
  b1_vec, above1 = _rank_scan(h1v, R_LOW)
  lo = lax.shift_left(b1_vec, SHIFT1)
  lane = lax.iota(jnp.int32, L)
  lanes = [lane + h * H2_STRIDE for h in range(NHIST)]
  ones = jnp.ones((L,), jnp.int32)
  junk = jnp.full((L,), NB, jnp.int32)

  def process(buf, carry):
    @plsc.parallel_loop(0, CHUNK, step=L * NHIST, unroll=2)
    def _(i):
      for h in range(NHIST):
        x = buf[pl.ds(i + h * L, L)]
        d = plsc.bitcast(x, jnp.int32) - lo
        b = jnp.minimum(lax.shift_right_logical(d, SHIFT2), junk)
        idx = lax.shift_left(b, 4) + lanes[h]
        plsc.addupdate_scatter(hist, [idx], ones)

    return carry

  _stream_chunks(gray_hbm, wid * PER_SUB, buf0, buf1, sem0, sem1, process, 0)
  _reduce_hists(hist, red, H2_STRIDE, H2_STRIDE)
  pltpu.sync_copy(red, out_hbm.at[wid])
  metav[pl.ds(0, L)] = jnp.zeros((L,), jnp.int32) + above1
  metav[pl.ds(L, L)] = lo
  pltpu.sync_copy(metav, meta_hbm.at[wid])


@functools.partial(
    pl.kernel,
    mesh=_mesh,
    out_type=(
        jax.ShapeDtypeStruct((NSUB, H2_STRIDE), jnp.int32),
        jax.ShapeDtypeStruct((NSUB, L), jnp.uint32),
        jax.ShapeDtypeStruct((NSUB, 2 * L), jnp.int32),
    ),
    scratch_types=[
        pltpu.VMEM((CHUNK,), jnp.float32),
        pltpu.VMEM((CHUNK,), jnp.float32),
        pltpu.VMEM((NB,), jnp.int32),
        pltpu.VMEM((2 * L,), jnp.int32),
        pltpu.VMEM((NHIST * H2_STRIDE,), jnp.int32),
        pltpu.VMEM((H2_STRIDE,), jnp.int32),
        pltpu.VMEM((L,), jnp.uint32),
        pltpu.VMEM((2 * L,), jnp.int32),
        pltpu.SemaphoreType.DMA,
        pltpu.SemaphoreType.DMA,
    ],
    compiler_params=_sc_params,
)
def _sc_round3(gray_hbm, h2_hbm, meta2_hbm, out_hbm, min_hbm, meta_hbm,
               buf0, buf1, h2v, m2v, hist, red, minv, metav, sem0, sem1):
  wid = lax.axis_index("s") * 2 + lax.axis_index("c")
  pltpu.sync_copy(h2_hbm, h2v)
  pltpu.sync_copy(meta2_hbm, m2v)
  _zero_hist(hist, NHIST * H2_STRIDE)
  above1 = lax.shift_right_arithmetic(jnp.sum(m2v[pl.ds(0, L)]), 4)
  lo1_vec = m2v[pl.ds(L, L)]
  b2_vec, above_r2 = _rank_scan(h2v, R_LOW - above1)
  above2 = above1 + above_r2
  lo2_vec = lo1_vec + lax.shift_left(b2_vec, SHIFT2)
  lo = plsc.bitcast(lo2_vec, jnp.uint32)
  lane = lax.iota(jnp.int32, L)
  lanes = [lane + h * H2_STRIDE for h in range(NHIST)]
  ones = jnp.ones((L,), jnp.int32)
  junk = jnp.full((L,), NB, jnp.uint32)
  u_init = jnp.full((L,), 0xFFFFFFFF, jnp.uint32)

  def process(buf, acc):
    @plsc.parallel_loop(0, CHUNK, step=L * NHIST, unroll=2, carry=acc)
    def body(i, acc2):
      for h in range(NHIST):
        x = buf[pl.ds(i + h * L, L)]
        d = plsc.bitcast(x, jnp.uint32) - lo
        b = plsc.bitcast(jnp.minimum(d, junk), jnp.int32)
        idx = lax.shift_left(b, 4) + lanes[h]
        plsc.addupdate_scatter(hist, [idx], ones)
        acc2 = jnp.minimum(acc2, d - junk)
      return acc2

    return body

  acc = _stream_chunks(
      gray_hbm, wid * PER_SUB, buf0, buf1, sem0, sem1, process, u_init)
  minv[...] = acc
  _reduce_hists(hist, red, H2_STRIDE, H2_STRIDE)
  pltpu.sync_copy(red, out_hbm.at[wid])
  pltpu.sync_copy(minv, min_hbm.at[wid])
  metav[pl.ds(0, L)] = jnp.zeros((L,), jnp.int32) + above2
  metav[pl.ds(L, L)] = lo2_vec
  pltpu.sync_copy(metav, meta_hbm.at[wid])


def _cmp_body(t_ref, gray_ref, out_ref):
  mask = (gray_ref[...] >= t_ref[0, 0]).astype(jnp.int32)
  out_ref[...] = mask.reshape(1, 1, H, W)


def _compare(gray, t):
  return pl.pallas_call(
      _cmp_body,
      out_shape=jax.ShapeDtypeStruct((B, 1, H, W), jnp.int32),
      grid=(B,),
      in_specs=[
          pl.BlockSpec(memory_space=pltpu.SMEM),
          pl.BlockSpec((H * W,), lambda i: (i,)),
      ],
      out_specs=pl.BlockSpec((1, 1, H, W), lambda i: (i, 0, 0, 0)),
  )(t.reshape(1, 1), gray)


def _suffix_count(hist):
  s = jnp.cumsum(hist[::-1])[::-1]
  return jnp.concatenate([s, jnp.zeros((1,), s.dtype)])


def kernel(img):
  flat = _grayscale(img)

  h1 = jnp.sum(_sc_round1(flat).reshape(NSUB, NB, L), axis=(0, 2))

  hist2, meta2 = _sc_round2(flat, h1)
  h2 = jnp.sum(hist2.reshape(NSUB, H2_BINS, L), axis=(0, 2))[:NB]

  hist3, mins, meta3 = _sc_round3(flat, h2, meta2[0])
  h3 = jnp.sum(hist3.reshape(NSUB, H2_BINS, L), axis=(0, 2))[:NB]
  s3 = _suffix_count(h3)
  above2 = meta3[0, 0]
  lo2 = meta3[0, L]

  p_lo = jnp.sum((s3[:NB] >= (R_LOW - above2)).astype(jnp.int32)) - 1
  v_low = lax.bitcast_convert_type((lo2 + p_lo).astype(jnp.int32), jnp.float32)
  p_hi = jnp.sum((s3[:NB] >= (R_HIGH - above2)).astype(jnp.int32)) - 1
  min_above_bits = (lo2 + NB).astype(jnp.uint32) + jnp.min(mins)
  v_high = jnp.where(
      above2 >= R_HIGH,
      lax.bitcast_convert_type(min_above_bits, jnp.float32),
      lax.bitcast_convert_type((lo2 + p_hi).astype(jnp.int32), jnp.float32))
  t = v_low * 0.75 + v_high * 0.25

  return _compare(flat, t)


# speedup vs baseline: 1.0623x; 1.0623x over previous
import functools

import jax
import jax.numpy as jnp
from jax import lax
from jax.experimental import pallas as pl
from jax.experimental.pallas import tpu as pltpu
from jax.experimental.pallas import tpu_sc as plsc

B, C, H, W = 16, 3, 512, 512
N = B * H * W
NSUB = 32
PER_SUB = N // NSUB
CHUNK = 16384
NCHUNK = PER_SUB // CHUNK
L = 16
UNROLL = 8
NHIST = 4
NB = 1024
SHIFT1 = 20
SHIFT2 = 10
H1_STRIDE = NB * L
H2_BINS = 1032
H2_STRIDE = H2_BINS * L
R_HIGH = 10486
R_LOW = 10487

_mesh = plsc.VectorSubcoreMesh(
    core_axis_name="c", subcore_axis_name="s", num_cores=2, num_subcores=16
)
_sc_params = pltpu.CompilerParams(needs_layout_passes=False)


def _gray_body(img_ref, out_ref):
  r = img_ref[0, 0]
  g = img_ref[0, 1]
  b = img_ref[0, 2]
  out_ref[...] = (0.2989 * r + 0.587 * g + 0.114 * b).reshape(H * W)


def _grayscale(img):
  return pl.pallas_call(
      _gray_body,
      out_shape=jax.ShapeDtypeStruct((N,), jnp.float32),
      grid=(B,),
      in_specs=[pl.BlockSpec((1, C, H, W), lambda i: (i, 0, 0, 0))],
      out_specs=pl.BlockSpec((H * W,), lambda i: (i,)),
  )(img)


def _zero_hist(hist, nwords):
  zeros = jnp.zeros((L,), jnp.int32)

  @plsc.parallel_loop(0, nwords, step=L * UNROLL, unroll=4)
  def _(i):
    for u in range(UNROLL):
      hist[pl.ds(i + u * L, L)] = zeros


def _reduce_hists(hist, out, stride, nwords):
  @plsc.parallel_loop(0, nwords, step=L, unroll=4)
  def _(i):
    acc = hist[pl.ds(i, L)]
    for h in range(1, NHIST):
      acc = acc + hist[pl.ds(h * stride + i, L)]
    out[pl.ds(i, L)] = acc


def _stream_chunks(gray_hbm, base, buf0, buf1, sem0, sem1, process, carry):
  pltpu.async_copy(gray_hbm.at[pl.ds(base, CHUNK)], buf0, sem0)
  pltpu.async_copy(gray_hbm.at[pl.ds(base + CHUNK, CHUNK)], buf1, sem1)

  def wait(buf, sem):
    pltpu.make_async_copy(gray_hbm.at[pl.ds(0, CHUNK)], buf, sem).wait()

  def body(i, c2):
    c = 2 * i
    wait(buf0, sem0)
    c2 = process(buf0, c2)
    pltpu.async_copy(
        gray_hbm.at[pl.ds(base + (c + 2) * CHUNK, CHUNK)], buf0, sem0)
    wait(buf1, sem1)
    c2 = process(buf1, c2)
    pltpu.async_copy(
        gray_hbm.at[pl.ds(base + (c + 3) * CHUNK, CHUNK)], buf1, sem1)
    return c2

  carry = lax.fori_loop(0, NCHUNK // 2 - 1, body, carry)
  wait(buf0, sem0)
  carry = process(buf0, carry)
  wait(buf1, sem1)
  carry = process(buf1, carry)
  return carry


@functools.partial(
    pl.kernel,
    mesh=_mesh,
    out_type=jax.ShapeDtypeStruct((NSUB, H1_STRIDE), jnp.int32),
    scratch_types=[
        pltpu.VMEM((CHUNK,), jnp.float32),
        pltpu.VMEM((CHUNK,), jnp.float32),
        pltpu.VMEM((NHIST * H1_STRIDE,), jnp.int32),
        pltpu.VMEM((H1_STRIDE,), jnp.int32),
        pltpu.SemaphoreType.DMA,
        pltpu.SemaphoreType.DMA,
    ],
    compiler_params=_sc_params,
)
def _sc_round1(gray_hbm, out_hbm, buf0, buf1, hist, red, sem0, sem1):
  wid = lax.axis_index("s") * 2 + lax.axis_index("c")
  _zero_hist(hist, NHIST * H1_STRIDE)
  lane = lax.iota(jnp.int32, L)
  lanes = [lane + h * H1_STRIDE for h in range(NHIST)]
  ones = jnp.ones((L,), jnp.int32)

  def process(buf, carry):
    @plsc.parallel_loop(0, CHUNK, step=L * NHIST, unroll=2)
    def _(i):
      for h in range(NHIST):
        x = buf[pl.ds(i + h * L, L)]
        bits = plsc.bitcast(x, jnp.int32)
        idx = lax.shift_left(
            lax.shift_right_logical(bits, SHIFT1), 4) + lanes[h]
        plsc.addupdate_scatter(hist, [idx], ones)

    return carry

  _stream_chunks(gray_hbm, wid * PER_SUB, buf0, buf1, sem0, sem1, process, 0)
  _reduce_hists(hist, red, H1_STRIDE, H1_STRIDE)
  pltpu.sync_copy(red, out_hbm.at[wid])


@functools.partial(
    pl.kernel,
    mesh=_mesh,
    out_type=jax.ShapeDtypeStruct((NSUB, H2_STRIDE), jnp.int32),
    scratch_types=[
        pltpu.VMEM((CHUNK,), jnp.float32),
        pltpu.VMEM((CHUNK,), jnp.float32),
        pltpu.VMEM((L,), jnp.int32),
        pltpu.VMEM((NHIST * H2_STRIDE,), jnp.int32),
        pltpu.VMEM((H2_STRIDE,), jnp.int32),
        pltpu.SemaphoreType.DMA,
        pltpu.SemaphoreType.DMA,
    ],
    compiler_params=_sc_params,
)
def _sc_round2(gray_hbm, lo_hbm, out_hbm,
               buf0, buf1, lov, hist, red, sem0, sem1):
  wid = lax.axis_index("s") * 2 + lax.axis_index("c")
  pltpu.sync_copy(lo_hbm, lov)
  _zero_hist(hist, NHIST * H2_STRIDE)
  lo = lov[...]
  lane = lax.iota(jnp.int32, L)
  lanes = [lane + h * H2_STRIDE for h in range(NHIST)]
  ones = jnp.ones((L,), jnp.int32)
  junk = jnp.full((L,), NB, jnp.int32)

  def process(buf, carry):
    @plsc.parallel_loop(0, CHUNK, step=L * NHIST, unroll=2)
    def _(i):
      for h in range(NHIST):
        x = buf[pl.ds(i + h * L, L)]
        d = plsc.bitcast(x, jnp.int32) - lo
        b = jnp.minimum(lax.shift_right_logical(d, SHIFT2), junk)
        idx = lax.shift_left(b, 4) + lanes[h]
        plsc.addupdate_scatter(hist, [idx], ones)

    return carry

  _stream_chunks(gray_hbm, wid * PER_SUB, buf0, buf1, sem0, sem1, process, 0)
  _reduce_hists(hist, red, H2_STRIDE, H2_STRIDE)
  pltpu.sync_copy(red, out_hbm.at[wid])


@functools.partial(
    pl.kernel,
    mesh=_mesh,
    out_type=(
        jax.ShapeDtypeStruct((NSUB, H2_STRIDE), jnp.int32),
        jax.ShapeDtypeStruct((NSUB, L), jnp.uint32),
    ),
    scratch_types=[
        pltpu.VMEM((CHUNK,), jnp.float32),
        pltpu.VMEM((CHUNK,), jnp.float32),
        pltpu.VMEM((L,), jnp.int32),
        pltpu.VMEM((NHIST * H2_STRIDE,), jnp.int32),
        pltpu.VMEM((H2_STRIDE,), jnp.int32),
        pltpu.VMEM((L,), jnp.uint32),
        pltpu.SemaphoreType.DMA,
        pltpu.SemaphoreType.DMA,
    ],
    compiler_params=_sc_params,
)
def _sc_round3(gray_hbm, lo_hbm, out_hbm, min_hbm,
               buf0, buf1, lov, hist, red, minv, sem0, sem1):
  wid = lax.axis_index("s") * 2 + lax.axis_index("c")
  pltpu.sync_copy(lo_hbm, lov)
  _zero_hist(hist, NHIST * H2_STRIDE)
  lo = plsc.bitcast(lov[...], jnp.uint32)
  lane = lax.iota(jnp.int32, L)
  lanes = [lane + h * H2_STRIDE for h in range(NHIST)]
  ones = jnp.ones((L,), jnp.int32)
  junk = jnp.full((L,), NB, jnp.uint32)
  u_init = jnp.full((L,), 0xFFFFFFFF, jnp.uint32)

  def process(buf, acc):
    @plsc.parallel_loop(0, CHUNK, step=L * NHIST, unroll=2, carry=acc)
    def body(i, acc2):
      for h in range(NHIST):
        x = buf[pl.ds(i + h * L, L)]
        d = plsc.bitcast(x, jnp.uint32) - lo
        b = plsc.bitcast(jnp.minimum(d, junk), jnp.int32)
        idx = lax.shift_left(b, 4) + lanes[h]
        plsc.addupdate_scatter(hist, [idx], ones)
        acc2 = jnp.minimum(acc2, d - junk)
      return acc2

    return body

  acc = _stream_chunks(
      gray_hbm, wid * PER_SUB, buf0, buf1, sem0, sem1, process, u_init)
  minv[...] = acc
  _reduce_hists(hist, red, H2_STRIDE, H2_STRIDE)
  pltpu.sync_copy(red, out_hbm.at[wid])
  pltpu.sync_copy(minv, min_hbm.at[wid])


def _cmp_body(t_ref, gray_ref, out_ref):
  mask = (gray_ref[...] >= t_ref[0, 0]).astype(jnp.int32)
  out_ref[...] = mask.reshape(1, 1, H, W)


def _compare(gray, t):
  return pl.pallas_call(
      _cmp_body,
      out_shape=jax.ShapeDtypeStruct((B, 1, H, W), jnp.int32),
      grid=(B,),
      in_specs=[
          pl.BlockSpec(memory_space=pltpu.SMEM),
          pl.BlockSpec((H * W,), lambda i: (i,)),
      ],
      out_specs=pl.BlockSpec((1, 1, H, W), lambda i: (i, 0, 0, 0)),
  )(t.reshape(1, 1), gray)


def _suffix_count(hist):
  s = jnp.cumsum(hist[::-1])[::-1]
  return jnp.concatenate([s, jnp.zeros((1,), s.dtype)])


def _bcast(v):
  return jnp.full((L,), v, dtype=jnp.int32)


def kernel(img):
  flat = _grayscale(img)

  h1 = jnp.sum(_sc_round1(flat).reshape(NSUB, NB, L), axis=(0, 2))
  s1 = _suffix_count(h1)
  b1 = jnp.sum((s1[:NB] >= R_LOW).astype(jnp.int32)) - 1
  above1 = s1[b1 + 1]
  lo1 = b1 << SHIFT1

  h2 = jnp.sum(
      _sc_round2(flat, _bcast(lo1)).reshape(NSUB, H2_BINS, L), axis=(0, 2)
  )[:NB]
  s2 = _suffix_count(h2)
  b2 = jnp.sum((s2[:NB] >= (R_LOW - above1)).astype(jnp.int32)) - 1
  above2 = above1 + s2[b2 + 1]
  lo2 = lo1 + (b2 << SHIFT2)

  hist3, mins = _sc_round3(flat, _bcast(lo2))
  h3 = jnp.sum(hist3.reshape(NSUB, H2_BINS, L), axis=(0, 2))[:NB]
  s3 = _suffix_count(h3)

  p_lo = jnp.sum((s3[:NB] >= (R_LOW - above2)).astype(jnp.int32)) - 1
  v_low = lax.bitcast_convert_type((lo2 + p_lo).astype(jnp.int32), jnp.float32)
  p_hi = jnp.sum((s3[:NB] >= (R_HIGH - above2)).astype(jnp.int32)) - 1
  min_above_bits = (lo2 + NB).astype(jnp.uint32) + jnp.min(mins)
  v_high = jnp.where(
      above2 >= R_HIGH,
      lax.bitcast_convert_type(min_above_bits, jnp.float32),
      lax.bitcast_convert_type((lo2 + p_hi).astype(jnp.int32), jnp.float32))
  t = v_low * 0.75 + v_high * 0.25

  return _compare(flat, t)
